# baseline (speedup 1.0000x reference)
import functools

import jax
import jax.numpy as jnp
from jax import lax
from jax.experimental import pallas as pl
from jax.experimental.pallas import tpu as pltpu
from jax.experimental.pallas import tpu_sc as plsc

SEQ_LEN, BATCH, DIM = 200, 4096, 64
TOTAL = SEQ_LEN * BATCH
NC, NS = 2, 16
NW = NC * NS
CHUNK = 256
NBLOCK = TOTAL // CHUNK
B_PER_W = NBLOCK // NW
NPAIR = B_PER_W // 2
SUB = CHUNK // 128

_mesh = plsc.VectorSubcoreMesh(core_axis_name="c", subcore_axis_name="s")


@functools.partial(
    pl.kernel,
    out_type=jax.ShapeDtypeStruct((SEQ_LEN, DIM // 8, BATCH // 128, 8, 128),
                                  jnp.float32),
    mesh=_mesh,
    scratch_types=[
        pltpu.VMEM((CHUNK,), jnp.int32),
        pltpu.VMEM((CHUNK,), jnp.int32),
        pltpu.VMEM((CHUNK, DIM), jnp.float32),
        pltpu.VMEM((CHUNK, DIM), jnp.float32),
        pltpu.VMEM((DIM // 8, 8, CHUNK), jnp.float32),
        pltpu.VMEM((DIM // 8, 8, CHUNK), jnp.float32),
        pltpu.SemaphoreType.DMA,
        pltpu.SemaphoreType.DMA,
        pltpu.SemaphoreType.DMA,
        pltpu.SemaphoreType.DMA,
    ],
    compiler_params=pltpu.CompilerParams(use_tc_tiling_on_sc=False,
                                         needs_layout_passes=False),
)
def _gather_kernel(idx_hbm, table_hbm, out_hbm,
                   idx0, idx1, rows0, rows1, t0, t1, sg0, sg1, so0, so1):
    wid = lax.axis_index("s") * NC + lax.axis_index("c")
    bbase = wid * B_PER_W

    def fetch(b, idx_v, rows_v, sg):
        off = pl.multiple_of(b * CHUNK, 8)
        pltpu.sync_copy(idx_hbm.at[pl.ds(off, CHUNK)], idx_v)
        pltpu.make_async_copy(table_hbm.at[idx_v], rows_v, sg).start()

    def wait_gather(idx_v, rows_v, sg):
        pltpu.make_async_copy(table_hbm.at[idx_v], rows_v, sg).wait()

    def transpose(rows_v, t_v):
        def jbody(jb, carry):
            jvec = jb * 16 + lax.iota(jnp.int32, 16)
            for g in range(DIM // 16):
                vals = [plsc.load_gather(rows_v, [jvec, jnp.full((16,), f, jnp.int32)])
                        for f in range(g * 16, (g + 1) * 16)]
                for k, f in enumerate(range(g * 16, (g + 1) * 16)):
                    t_v[f // 8, f % 8, pl.ds(jb * 16, 16)] = vals[k]
            return carry

        lax.fori_loop(0, CHUNK // 16, jbody, 0)

    def wb(b, t_v, so, start):
        tmp = b // 4
        rq = b - tmp * 4
        tc = tmp % 32
        tr = tmp // 32
        for rr in range(SUB):
            s = tr * 8 + rq * 2 + rr
            src = t_v.at[:, :, pl.ds(rr * 128, 128)]
            dst = out_hbm.at[s, :, tc, :, :]
            cp = pltpu.make_async_copy(src, dst, so)
            if start:
                cp.start()
            else:
                cp.wait()

    fetch(bbase, idx0, rows0, sg0)
    fetch(bbase + 1, idx1, rows1, sg1)
    wait_gather(idx0, rows0, sg0)
    transpose(rows0, t0)
    wb(bbase, t0, so0, True)
    wait_gather(idx1, rows1, sg1)
    transpose(rows1, t1)
    wb(bbase + 1, t1, so1, True)

    def body(i, carry):
        b0 = bbase + i * 2
        b1 = b0 + 1
        fetch(b0, idx0, rows0, sg0)
        fetch(b1, idx1, rows1, sg1)
        wait_gather(idx0, rows0, sg0)
        wb(b0 - 2, t0, so0, False)
        transpose(rows0, t0)
        wb(b0, t0, so0, True)
        wait_gather(idx1, rows1, sg1)
        wb(b1 - 2, t1, so1, False)
        transpose(rows1, t1)
        wb(b1, t1, so1, True)
        return carry

    lax.fori_loop(1, NPAIR, body, 0)
    wb(bbase + B_PER_W - 2, t0, so0, False)
    wb(bbase + B_PER_W - 1, t1, so1, False)


def kernel(input, weight):
    idx = (input.astype(jnp.int32).reshape(25, 8, 32, 128)
           .transpose(0, 2, 1, 3).reshape(-1))
    out = _gather_kernel(idx, weight)
    return out.transpose(0, 2, 4, 1, 3).reshape(SEQ_LEN, BATCH, DIM)


# speedup vs baseline: 1.1022x; 1.1022x over previous
import functools

import jax
import jax.numpy as jnp
from jax import lax
from jax.experimental import pallas as pl
from jax.experimental.pallas import tpu as pltpu
from jax.experimental.pallas import tpu_sc as plsc

SEQ_LEN, BATCH, DIM = 200, 4096, 64
TOTAL = SEQ_LEN * BATCH
NC, NS = 2, 16
NW = NC * NS
CHUNK = 256
NBLOCK = TOTAL // CHUNK
B_PER_W = NBLOCK // NW
NPAIR = B_PER_W // 2
SUB = CHUNK // 128

_mesh = plsc.VectorSubcoreMesh(core_axis_name="c", subcore_axis_name="s")


@functools.partial(
    pl.kernel,
    out_type=jax.ShapeDtypeStruct((SEQ_LEN, DIM // 8, BATCH // 128, 8, 128),
                                  jnp.float32),
    mesh=_mesh,
    scratch_types=[
        pltpu.VMEM((CHUNK,), jnp.int32),
        pltpu.VMEM((CHUNK,), jnp.int32),
        pltpu.VMEM((CHUNK, DIM), jnp.float32),
        pltpu.VMEM((CHUNK, DIM), jnp.float32),
        pltpu.VMEM((DIM // 8, 8, CHUNK), jnp.float32),
        pltpu.VMEM((DIM // 8, 8, CHUNK), jnp.float32),
        pltpu.SemaphoreType.DMA,
        pltpu.SemaphoreType.DMA,
        pltpu.SemaphoreType.DMA,
        pltpu.SemaphoreType.DMA,
    ],
    compiler_params=pltpu.CompilerParams(use_tc_tiling_on_sc=False,
                                         needs_layout_passes=False),
)
def _gather_kernel(idx_hbm, table_hbm, out_hbm,
                   idx0, idx1, rows0, rows1, t0, t1, sg0, sg1, so0, so1):
    wid = lax.axis_index("s") * NC + lax.axis_index("c")
    bbase = wid * B_PER_W

    def fetch(b, idx_v, rows_v, sg):
        off = pl.multiple_of(b * CHUNK, 8)
        pltpu.sync_copy(idx_hbm.at[pl.ds(off, CHUNK)], idx_v)
        pltpu.make_async_copy(table_hbm.at[idx_v], rows_v, sg).start()

    def wait_gather(idx_v, rows_v, sg):
        pltpu.make_async_copy(table_hbm.at[idx_v], rows_v, sg).wait()

    def transpose(rows_v, t_v):
        def jbody(jb, carry):
            lanes = lax.iota(jnp.int32, 16)
            jvec = jb * 16 + lanes
            for f0 in range(0, DIM, 16):
                for k in range(16):
                    fv = f0 + ((lanes + k) & 15)
                    vals = plsc.load_gather(rows_v, [jvec, fv])
                    plsc.store_scatter(t_v, [fv >> 3, fv & 7, jvec], vals)
            return carry

        lax.fori_loop(0, CHUNK // 16, jbody, 0)

    def wb(b, t_v, so, start):
        tmp = b // 4
        rq = b - tmp * 4
        tc = tmp % 32
        tr = tmp // 32
        for rr in range(SUB):
            s = tr * 8 + rq * 2 + rr
            src = t_v.at[:, :, pl.ds(rr * 128, 128)]
            dst = out_hbm.at[s, :, tc, :, :]
            cp = pltpu.make_async_copy(src, dst, so)
            if start:
                cp.start()
            else:
                cp.wait()

    fetch(bbase, idx0, rows0, sg0)
    fetch(bbase + 1, idx1, rows1, sg1)
    wait_gather(idx0, rows0, sg0)
    transpose(rows0, t0)
    wb(bbase, t0, so0, True)
    wait_gather(idx1, rows1, sg1)
    transpose(rows1, t1)
    wb(bbase + 1, t1, so1, True)

    def body(i, carry):
        b0 = bbase + i * 2
        b1 = b0 + 1
        fetch(b0, idx0, rows0, sg0)
        fetch(b1, idx1, rows1, sg1)
        wait_gather(idx0, rows0, sg0)
        wb(b0 - 2, t0, so0, False)
        transpose(rows0, t0)
        wb(b0, t0, so0, True)
        wait_gather(idx1, rows1, sg1)
        wb(b1 - 2, t1, so1, False)
        transpose(rows1, t1)
        wb(b1, t1, so1, True)
        return carry

    lax.fori_loop(1, NPAIR, body, 0)
    wb(bbase + B_PER_W - 2, t0, so0, False)
    wb(bbase + B_PER_W - 1, t1, so1, False)


def kernel(input, weight):
    idx = (input.astype(jnp.int32).reshape(25, 8, 32, 128)
           .transpose(0, 2, 1, 3).reshape(-1))
    out = _gather_kernel(idx, weight)
    return out.transpose(0, 2, 4, 1, 3).reshape(SEQ_LEN, BATCH, DIM)
